# baseline (speedup 1.0000x reference)
import functools

import jax
import jax.numpy as jnp
from jax import lax
from jax.experimental import pallas as pl
from jax.experimental.pallas import tpu as pltpu
from jax.experimental.pallas import tpu_sc as plsc

B = 16384
NCLS = 1000
NC, NS, L = 2, 16, 16
NW = NC * NS
CPW = B // NW
PC = 64
PARTS = tuple(
    (p * PC, min((p + 1) * PC, NCLS)) for p in range((NCLS + PC - 1) // PC)
)
NBUF = 3
CW = PC * CPW

_mesh = plsc.VectorSubcoreMesh(core_axis_name="c", subcore_axis_name="s")


@functools.partial(
    pl.kernel,
    out_type=jax.ShapeDtypeStruct((NCLS, B), jnp.float32),
    mesh=_mesh,
    compiler_params=pltpu.CompilerParams(
        needs_layout_passes=False,
        use_tc_tiling_on_sc=True,
    ),
    scratch_types=[
        pltpu.VMEM((CPW,), jnp.int32),
        *([pltpu.VMEM((PC, CPW), jnp.float32)] * NBUF),
        *([pltpu.SemaphoreType.DMA] * NBUF),
    ],
)
def _onehot_sc(idx_hbm, out_hbm, idx_v, *bufs_sems):
    bufs = bufs_sems[:NBUF]
    sems = bufs_sems[NBUF:]
    wid = lax.axis_index("s") * NC + lax.axis_index("c")
    col0 = wid * CPW

    pltpu.sync_copy(idx_hbm.at[pl.ds(col0, CPW)], idx_v)

    lane = lax.iota(jnp.int32, L)
    ones = jnp.ones((L,), jnp.float32)
    zvec = jnp.zeros((L,), jnp.float32)

    def _zero(buf):
        def _zbody(i, carry):
            r = i // (CPW // L)
            c = (i % (CPW // L)) * L
            buf[r, pl.ds(c, L)] = zvec
            return carry

        lax.fori_loop(0, CW // L, _zbody, 0)

    def _scatter(buf, p, val):
        row_lo, row_hi = PARTS[p]
        for j in range(CPW // L):
            idxv = idx_v[pl.ds(j * L, L)]
            rows = idxv - row_lo
            cols = lane + j * L
            m = (idxv >= row_lo) & (idxv < row_hi)
            plsc.store_scatter(buf, [rows, cols], val, mask=m)

    pending = [None] * NBUF
    for p in range(len(PARTS)):
        b = p % NBUF
        buf = bufs[b]
        if pending[b] is None:
            _zero(buf)
        else:
            op, h = pending[b]
            h.wait()
            _scatter(buf, op, zvec)
        _scatter(buf, p, ones)
        row_lo, row_hi = PARTS[p]
        pr = row_hi - row_lo
        src = buf if pr == PC else buf.at[pl.ds(0, pr), :]
        h = pltpu.async_copy(
            src,
            out_hbm.at[pl.ds(row_lo, pr), pl.ds(col0, CPW)],
            sems[b],
        )
        pending[b] = (p, h)

    for b in range(NBUF):
        if pending[b] is not None:
            pending[b][1].wait()


def kernel(X_train):
    idx = X_train.reshape(B).astype(jnp.int32)
    return _onehot_sc(idx).T


# speedup vs baseline: 1.1329x; 1.1329x over previous
import functools

import jax
import jax.numpy as jnp
from jax import lax
from jax.experimental import pallas as pl
from jax.experimental.pallas import tpu as pltpu
from jax.experimental.pallas import tpu_sc as plsc

B = 16384
NCLS = 1000
NC, NS, L = 2, 16, 16
NW = NC * NS
CPW = B // NW
PC = 256
PB = 128
Q = CPW // PB
PARTS = ((0, 248), (248, 496), (496, 744), (744, 1000))
NBUF = 3
CW = PC * PB

_mesh = plsc.VectorSubcoreMesh(core_axis_name="c", subcore_axis_name="s")


@functools.partial(
    pl.kernel,
    out_type=jax.ShapeDtypeStruct((NCLS, B), jnp.float32),
    mesh=_mesh,
    compiler_params=pltpu.CompilerParams(
        needs_layout_passes=False,
        use_tc_tiling_on_sc=True,
    ),
    scratch_types=[
        pltpu.VMEM((CPW,), jnp.int32),
        *([pltpu.VMEM((PC, PB), jnp.float32)] * NBUF),
        *([pltpu.SemaphoreType.DMA] * NBUF),
    ],
)
def _onehot_sc(idx_hbm, out_hbm, idx_v, *bufs_sems):
    bufs = bufs_sems[:NBUF]
    sems = bufs_sems[NBUF:]
    wid = lax.axis_index("s") * NC + lax.axis_index("c")
    col0 = wid * CPW

    pltpu.sync_copy(idx_hbm.at[pl.ds(col0, CPW)], idx_v)

    lane = lax.iota(jnp.int32, L)
    ones = jnp.ones((L,), jnp.float32)
    zvec = jnp.zeros((L,), jnp.float32)

    def _zero(buf):
        def _zbody(i, carry):
            r = i // (PB // L)
            c = (i % (PB // L)) * L
            buf[r, pl.ds(c, L)] = zvec
            return carry

        lax.fori_loop(0, CW // L, _zbody, 0)

    def _scatter(buf, q, p, val):
        row_lo, row_hi = PARTS[p]
        for j in range(PB // L):
            idxv = idx_v[pl.ds(q * PB + j * L, L)]
            rows = idxv - row_lo
            cols = lane + j * L
            m = (idxv >= row_lo) & (idxv < row_hi)
            plsc.store_scatter(buf, [rows, cols], val, mask=m)

    pending = [None] * NBUF
    chunks = [(q, p) for q in range(Q) for p in range(len(PARTS))]
    for n, (q, p) in enumerate(chunks):
        b = n % NBUF
        buf = bufs[b]
        if pending[b] is None:
            _zero(buf)
        else:
            oq, op, h = pending[b]
            h.wait()
            _scatter(buf, oq, op, zvec)
        _scatter(buf, q, p, ones)
        row_lo, row_hi = PARTS[p]
        pr = row_hi - row_lo
        src = buf if pr == PC else buf.at[pl.ds(0, pr), :]
        h = pltpu.async_copy(
            src,
            out_hbm.at[pl.ds(row_lo, pr), pl.ds(col0 + q * PB, PB)],
            sems[b],
        )
        pending[b] = (q, p, h)

    for b in range(NBUF):
        if pending[b] is not None:
            pending[b][2].wait()


def kernel(X_train):
    idx = X_train.reshape(B).astype(jnp.int32)
    return _onehot_sc(idx).T


# speedup vs baseline: 1.7108x; 1.5101x over previous
import functools

import jax
import jax.numpy as jnp
from jax import lax
from jax.experimental import pallas as pl
from jax.experimental.pallas import tpu as pltpu
from jax.experimental.pallas import tpu_sc as plsc

B = 16384
NCLS = 1000
NC, NS, L = 2, 16, 16
NW = NC * NS
CPW = B // NW
PC = 256
PB = 128
Q = CPW // PB
PARTS = ((0, 248), (248, 496), (496, 744), (744, 1000))
NBUF = 3
CW = PC * PB

_mesh = plsc.VectorSubcoreMesh(core_axis_name="c", subcore_axis_name="s")


@functools.partial(
    pl.kernel,
    out_type=jax.ShapeDtypeStruct((NCLS, B), jnp.float32),
    mesh=_mesh,
    compiler_params=pltpu.CompilerParams(
        needs_layout_passes=False,
        use_tc_tiling_on_sc=True,
    ),
    scratch_types=[
        pltpu.VMEM((CPW,), jnp.int32),
        *([pltpu.VMEM((PC, PB), jnp.float32)] * NBUF),
        *([pltpu.SemaphoreType.DMA] * NBUF),
    ],
)
def _onehot_sc(idx_hbm, out_hbm, idx_v, *bufs_sems):
    bufs = bufs_sems[:NBUF]
    sems = bufs_sems[NBUF:]
    wid = lax.axis_index("s") * NC + lax.axis_index("c")
    col0 = wid * CPW

    pltpu.sync_copy(idx_hbm.at[pl.ds(col0, CPW)], idx_v)

    lane = lax.iota(jnp.int32, L)
    ones = jnp.ones((L,), jnp.float32)
    zvec = jnp.zeros((L,), jnp.float32)

    def _zero(buf):
        def _zbody(r, carry):
            for j in range(PB // L):
                buf[r, pl.ds(j * L, L)] = zvec
            return carry

        lax.fori_loop(0, PC, _zbody, 0)

    def _scatter(buf, q, p, val):
        row_lo, row_hi = PARTS[p]
        for j in range(PB // L):
            idxv = idx_v[pl.ds(q * PB + j * L, L)]
            rows = idxv - row_lo
            cols = lane + j * L
            m = (idxv >= row_lo) & (idxv < row_hi)
            plsc.store_scatter(buf, [rows, cols], val, mask=m)

    pending = [None] * NBUF
    chunks = [(q, p) for q in range(Q) for p in range(len(PARTS))]
    for n, (q, p) in enumerate(chunks):
        b = n % NBUF
        buf = bufs[b]
        if pending[b] is None:
            _zero(buf)
        else:
            oq, op, h = pending[b]
            h.wait()
            _scatter(buf, oq, op, zvec)
        _scatter(buf, q, p, ones)
        row_lo, row_hi = PARTS[p]
        pr = row_hi - row_lo
        src = buf if pr == PC else buf.at[pl.ds(0, pr), :]
        h = pltpu.async_copy(
            src,
            out_hbm.at[pl.ds(row_lo, pr), pl.ds(col0 + q * PB, PB)],
            sems[b],
        )
        pending[b] = (q, p, h)

    for b in range(NBUF):
        if pending[b] is not None:
            pending[b][2].wait()


def kernel(X_train):
    idx = X_train.reshape(B).astype(jnp.int32)
    return _onehot_sc(idx).T
